# Initial kernel scaffold; baseline (speedup 1.0000x reference)
import functools

import jax
import jax.numpy as jnp
from jax import lax
from jax.experimental import pallas as pl
from jax.experimental.pallas import tpu as pltpu
from jax.experimental.pallas import tpu_sc as plsc

NC, NS = 2, 16
NW = NC * NS
H_CHUNK = 1600
G_ROWS = 25
G_CHUNK = G_ROWS * 128
VT = 2000
EPS = 1e-5


def _sc_mesh():
    return plsc.VectorSubcoreMesh(core_axis_name="c", subcore_axis_name="s")


def _histogram(v2p, n_vox):
    n_pts = v2p.shape[0]
    n_chunks = n_pts // H_CHUNK
    chunks_per_w = (n_chunks + NW - 1) // NW

    @functools.partial(
        pl.kernel,
        out_type=jax.ShapeDtypeStruct((NW, n_vox), jnp.float32),
        mesh=_sc_mesh(),
        scratch_types=[pltpu.VMEM((n_vox,), jnp.float32),
                       pltpu.VMEM((H_CHUNK,), jnp.int32)])
    def hist_kernel(idx_hbm, hist_hbm, hist_v, idx_v):
        wid = lax.axis_index("s") * NC + lax.axis_index("c")

        @pl.loop(0, n_vox, step=16)
        def _(i):
            hist_v[pl.ds(i, 16)] = jnp.zeros((16,), jnp.float32)

        @pl.loop(0, chunks_per_w)
        def _(j):
            c = wid + NW * j

            @pl.when(c < n_chunks)
            def _():
                pltpu.sync_copy(idx_hbm.at[pl.ds(c * H_CHUNK, H_CHUNK)], idx_v)

                @pl.loop(0, H_CHUNK, step=16)
                def _(i):
                    x = idx_v[pl.ds(i, 16)]
                    cnt, last = plsc.scan_count(x)
                    plsc.addupdate_scatter(hist_v, [x],
                                           cnt.astype(jnp.float32), mask=last)

        pltpu.sync_copy(hist_v, hist_hbm.at[wid])

    return hist_kernel(v2p)


def _gather_rows(table, v2p, d):
    n_pts = v2p.shape[0]
    n_chunks = n_pts // G_CHUNK
    chunks_per_w = (n_chunks + NW - 1) // NW
    idx2 = v2p.reshape(n_pts // 128, 128)

    @functools.partial(
        pl.kernel,
        out_type=jax.ShapeDtypeStruct((n_pts, d), jnp.float32),
        mesh=_sc_mesh(),
        scratch_types=[pltpu.VMEM((G_ROWS, 128), jnp.int32),
                       pltpu.VMEM((G_CHUNK, d), jnp.float32),
                       pltpu.SemaphoreType.DMA])
    def gather_kernel(table_hbm, idx_hbm, out_hbm, idx_v, rows_v, sem):
        wid = lax.axis_index("s") * NC + lax.axis_index("c")

        @pl.loop(0, chunks_per_w)
        def _(j):
            c = wid + NW * j

            @pl.when(c < n_chunks)
            def _():
                pltpu.sync_copy(idx_hbm.at[pl.ds(c * G_ROWS, G_ROWS)], idx_v)

                @pl.loop(0, G_ROWS)
                def _(r):
                    pltpu.async_copy(table_hbm.at[idx_v.at[r]],
                                     rows_v.at[pl.ds(r * 128, 128)], sem)

                @pl.loop(0, G_ROWS)
                def _(r):
                    pltpu.make_async_copy(table_hbm.at[idx_v.at[r]],
                                          rows_v.at[pl.ds(r * 128, 128)],
                                          sem).wait()

                pltpu.sync_copy(rows_v, out_hbm.at[pl.ds(c * G_CHUNK, G_CHUNK)])

    return gather_kernel(table, idx2)


def _stats1(feat, hist, wc):
    n_vox, cin = feat.shape
    nt = n_vox // VT
    c1 = wc.shape[1]

    def body(feat_ref, hist_ref, wc_ref, s1_ref, s2_ref, c_ref):
        t = pl.program_id(0)
        c = jnp.sum(hist_ref[...], axis=0, keepdims=True)
        c_ref[...] = c[None]
        hv = jnp.dot(feat_ref[...].astype(jnp.bfloat16),
                     wc_ref[...].astype(jnp.bfloat16),
                     preferred_element_type=jnp.float32)
        p1 = jnp.dot(c, hv, preferred_element_type=jnp.float32)
        p2 = jnp.dot(c, hv * hv, preferred_element_type=jnp.float32)

        @pl.when(t == 0)
        def _():
            s1_ref[...] = jnp.zeros_like(s1_ref)
            s2_ref[...] = jnp.zeros_like(s2_ref)

        s1_ref[...] += p1
        s2_ref[...] += p2

    return pl.pallas_call(
        body,
        grid=(nt,),
        in_specs=[
            pl.BlockSpec((VT, cin), lambda t: (t, 0)),
            pl.BlockSpec((NW, VT), lambda t: (0, t)),
            pl.BlockSpec((cin, c1), lambda t: (0, 0)),
        ],
        out_specs=[
            pl.BlockSpec((1, c1), lambda t: (0, 0)),
            pl.BlockSpec((1, c1), lambda t: (0, 0)),
            pl.BlockSpec((1, 1, VT), lambda t: (t, 0, 0)),
        ],
        out_shape=[
            jax.ShapeDtypeStruct((1, c1), jnp.float32),
            jax.ShapeDtypeStruct((1, c1), jnp.float32),
            jax.ShapeDtypeStruct((nt, 1, VT), jnp.float32),
        ],
    )(feat, hist, wc)


def _pass_b(feat, csum3, wc, w2, aff1_a, aff1_b, a1):
    n_vox, cin = feat.shape
    nt = n_vox // VT
    c1 = wc.shape[1]
    c2 = w2.shape[1]

    def body(feat_ref, c_ref, wc_ref, w2_ref, a_ref, b_ref, s_ref,
             h2_ref, t1_ref, t2_ref):
        t = pl.program_id(0)
        hv = jnp.dot(feat_ref[...].astype(jnp.bfloat16),
                     wc_ref[...].astype(jnp.bfloat16),
                     preferred_element_type=jnp.float32)
        y = hv * a_ref[...] + b_ref[...]
        z = jnp.where(y >= 0, y, s_ref[0, 0] * y)
        h2 = jnp.dot(z.astype(jnp.bfloat16), w2_ref[...].astype(jnp.bfloat16),
                     preferred_element_type=jnp.float32)
        h2_ref[...] = h2
        c = c_ref[0]
        p1 = jnp.dot(c, h2, preferred_element_type=jnp.float32)
        p2 = jnp.dot(c, h2 * h2, preferred_element_type=jnp.float32)

        @pl.when(t == 0)
        def _():
            t1_ref[...] = jnp.zeros_like(t1_ref)
            t2_ref[...] = jnp.zeros_like(t2_ref)

        t1_ref[...] += p1
        t2_ref[...] += p2

    return pl.pallas_call(
        body,
        grid=(nt,),
        in_specs=[
            pl.BlockSpec((VT, cin), lambda t: (t, 0)),
            pl.BlockSpec((1, 1, VT), lambda t: (t, 0, 0)),
            pl.BlockSpec((cin, c1), lambda t: (0, 0)),
            pl.BlockSpec((c1, c2), lambda t: (0, 0)),
            pl.BlockSpec((1, c1), lambda t: (0, 0)),
            pl.BlockSpec((1, c1), lambda t: (0, 0)),
            pl.BlockSpec((1, 1), lambda t: (0, 0)),
        ],
        out_specs=[
            pl.BlockSpec((VT, c2), lambda t: (t, 0)),
            pl.BlockSpec((1, c2), lambda t: (0, 0)),
            pl.BlockSpec((1, c2), lambda t: (0, 0)),
        ],
        out_shape=[
            jax.ShapeDtypeStruct((n_vox, c2), jnp.float32),
            jax.ShapeDtypeStruct((1, c2), jnp.float32),
            jax.ShapeDtypeStruct((1, c2), jnp.float32),
        ],
    )(feat, csum3, wc, w2, aff1_a, aff1_b, a1)


def _pass_c(h2, w3p, b3p, aff2_a, aff2_b, a2):
    n_vox, c2 = h2.shape
    nt = n_vox // VT
    d = w3p.shape[1]

    def body(h2_ref, w3_ref, b3_ref, a_ref, b_ref, s_ref, out_ref):
        y = h2_ref[...] * a_ref[...] + b_ref[...]
        z = jnp.where(y >= 0, y, s_ref[0, 0] * y)
        out_ref[...] = jnp.dot(z, w3_ref[...],
                               preferred_element_type=jnp.float32) + b3_ref[...]

    return pl.pallas_call(
        body,
        grid=(nt,),
        in_specs=[
            pl.BlockSpec((VT, c2), lambda t: (t, 0)),
            pl.BlockSpec((c2, d), lambda t: (0, 0)),
            pl.BlockSpec((1, d), lambda t: (0, 0)),
            pl.BlockSpec((1, c2), lambda t: (0, 0)),
            pl.BlockSpec((1, c2), lambda t: (0, 0)),
            pl.BlockSpec((1, 1), lambda t: (0, 0)),
        ],
        out_specs=pl.BlockSpec((VT, d), lambda t: (t, 0)),
        out_shape=jax.ShapeDtypeStruct((n_vox, d), jnp.float32),
    )(h2, w3p, b3p, aff2_a, aff2_b, a2)


def kernel(feat_voxel, xyz_voxel, v2p_v1, W_backbone, W1, g1, b1, a1,
           W2, g2, b2, a2, W3, b3):
    del xyz_voxel
    n_pts = v2p_v1.shape[0]
    n_vox = feat_voxel.shape[0]
    n_pts_f = jnp.float32(n_pts)

    wc = jnp.dot(W_backbone, W1, preferred_element_type=jnp.float32)
    d = W3.shape[1]
    b3p = b3.reshape(1, -1)

    hist = _histogram(v2p_v1, n_vox)

    s1, s2, csum3 = _stats1(feat_voxel, hist, wc)
    mean1 = s1 / n_pts_f
    var1 = s2 / n_pts_f - mean1 * mean1
    aff1_a = g1.reshape(1, -1) / jnp.sqrt(var1 + EPS)
    aff1_b = b1.reshape(1, -1) - mean1 * aff1_a

    h2, t1, t2 = _pass_b(feat_voxel, csum3, wc, W2, aff1_a, aff1_b,
                         a1.reshape(1, 1))
    mean2 = t1 / n_pts_f
    var2 = t2 / n_pts_f - mean2 * mean2
    aff2_a = g2.reshape(1, -1) / jnp.sqrt(var2 + EPS)
    aff2_b = b2.reshape(1, -1) - mean2 * aff2_a

    table = _pass_c(h2, W3, b3p, aff2_a, aff2_b, a2.reshape(1, 1))

    return _gather_rows(table, v2p_v1, d)


# speedup vs baseline: 1.1478x; 1.1478x over previous
import dataclasses
import functools
import math

import jax
import jax.numpy as jnp
from jax import lax
from jax.experimental import pallas as pl
from jax.experimental.pallas import tpu as pltpu
from jax.experimental.pallas import tpu_sc as plsc

NC, NS = 2, 16
NW = NC * NS
H_CHUNK = 1600
G_ROWS = 64
G_CHUNK = G_ROWS * 128
VT = 2000
EPS = 1e-5


def _sc_mesh():
    return plsc.VectorSubcoreMesh(core_axis_name="c", subcore_axis_name="s")


def _sc_no_layout_params():
    cp = pltpu.CompilerParams()
    if "needs_layout_passes" in pltpu.CompilerParams.__dataclass_fields__:
        cp = dataclasses.replace(cp, needs_layout_passes=False)
    return cp


def _histogram(v2p, n_vox):
    n_pts = v2p.shape[0]
    n_chunks = n_pts // H_CHUNK
    chunks_per_w = (n_chunks + NW - 1) // NW

    @functools.partial(
        pl.kernel,
        out_type=jax.ShapeDtypeStruct((NW * n_vox,), jnp.float32),
        mesh=_sc_mesh(),
        compiler_params=_sc_no_layout_params(),
        scratch_types=[pltpu.VMEM((n_vox,), jnp.float32),
                       pltpu.VMEM((H_CHUNK,), jnp.int32)])
    def hist_kernel(idx_hbm, hist_hbm, hist_v, idx_v):
        wid = lax.axis_index("s") * NC + lax.axis_index("c")

        @pl.loop(0, n_vox, step=16)
        def _(i):
            hist_v[pl.ds(i, 16)] = jnp.zeros((16,), jnp.float32)

        @pl.loop(0, chunks_per_w)
        def _(j):
            c = wid + NW * j

            @pl.when(c < n_chunks)
            def _():
                pltpu.sync_copy(idx_hbm.at[pl.ds(c * H_CHUNK, H_CHUNK)], idx_v)

                @pl.loop(0, H_CHUNK, step=16)
                def _(i):
                    x = idx_v[pl.ds(i, 16)]
                    cnt, last = plsc.scan_count(x)
                    plsc.addupdate_scatter(hist_v, [x],
                                           cnt.astype(jnp.float32), mask=last)

        pltpu.sync_copy(hist_v, hist_hbm.at[pl.ds(wid * n_vox, n_vox)])

    return hist_kernel(v2p)


def _merge_counts(hist, n_vox):
    m_chunk = 2000
    n_chunks = n_vox // m_chunk
    chunks_per_w = (n_chunks + NW - 1) // NW

    @functools.partial(
        pl.kernel,
        out_type=jax.ShapeDtypeStruct((n_vox,), jnp.float32),
        mesh=_sc_mesh(),
        scratch_types=[pltpu.VMEM((m_chunk,), jnp.float32),
                       pltpu.VMEM((m_chunk,), jnp.float32)])
    def merge_kernel(hist_hbm, counts_hbm, acc_v, buf_v):
        wid = lax.axis_index("s") * NC + lax.axis_index("c")

        @pl.loop(0, chunks_per_w)
        def _(j):
            c = wid + NW * j

            @pl.when(c < n_chunks)
            def _():
                pltpu.sync_copy(hist_hbm.at[pl.ds(c * m_chunk, m_chunk)],
                                acc_v)

                @pl.loop(1, NW)
                def _(k):
                    pltpu.sync_copy(
                        hist_hbm.at[pl.ds(k * n_vox + c * m_chunk, m_chunk)],
                        buf_v)

                    @pl.loop(0, m_chunk, step=16)
                    def _(i):
                        acc_v[pl.ds(i, 16)] += buf_v[pl.ds(i, 16)]

                pltpu.sync_copy(acc_v, counts_hbm.at[pl.ds(c * m_chunk, m_chunk)])

    return merge_kernel(hist)


def _gather_elems(table_flat, idx2d):
    n_rows = idx2d.shape[0]
    n_elems = n_rows * 128
    n_chunks = n_rows // G_ROWS
    chunks_per_w = (n_chunks + NW - 1) // NW

    @functools.partial(
        pl.kernel,
        out_type=jax.ShapeDtypeStruct((n_elems,), jnp.float32),
        mesh=_sc_mesh(),
        scratch_types=[pltpu.VMEM((G_ROWS, 128), jnp.int32),
                       pltpu.VMEM((G_CHUNK,), jnp.float32),
                       pltpu.SemaphoreType.DMA])
    def gather_kernel(table_hbm, idx_hbm, out_hbm, idx_v, vals_v, sem):
        wid = lax.axis_index("s") * NC + lax.axis_index("c")

        @pl.loop(0, chunks_per_w)
        def _(j):
            c = wid + NW * j

            @pl.when(c < n_chunks)
            def _():
                pltpu.sync_copy(idx_hbm.at[pl.ds(c * G_ROWS, G_ROWS)], idx_v)

                @pl.loop(0, G_ROWS)
                def _(r):
                    pltpu.async_copy(
                        table_hbm.at[idx_v.at[r]],
                        vals_v.at[pl.ds(r * 128, 128)], sem)

                @pl.loop(0, G_ROWS)
                def _(r):
                    pltpu.make_async_copy(
                        table_hbm.at[idx_v.at[r]],
                        vals_v.at[pl.ds(r * 128, 128)], sem).wait()

                pltpu.sync_copy(vals_v, out_hbm.at[pl.ds(c * G_CHUNK, G_CHUNK)])

    return gather_kernel(table_flat, idx2d)


def _stats1(feat, counts3, wc):
    n_vox, cin = feat.shape
    nt = n_vox // VT
    c1 = wc.shape[1]

    def body(feat_ref, c_ref, wc_ref, s1_ref, s2_ref):
        t = pl.program_id(0)
        c = c_ref[0]
        hv = jnp.dot(feat_ref[...].astype(jnp.bfloat16),
                     wc_ref[...].astype(jnp.bfloat16),
                     preferred_element_type=jnp.float32)
        chv = c * hv
        p1 = jnp.sum(chv, axis=0, keepdims=True)
        p2 = jnp.sum(chv * hv, axis=0, keepdims=True)

        @pl.when(t == 0)
        def _():
            s1_ref[...] = jnp.zeros_like(s1_ref)
            s2_ref[...] = jnp.zeros_like(s2_ref)

        s1_ref[...] += p1
        s2_ref[...] += p2

    return pl.pallas_call(
        body,
        grid=(nt,),
        in_specs=[
            pl.BlockSpec((VT, cin), lambda t: (t, 0)),
            pl.BlockSpec((1, VT, 1), lambda t: (t, 0, 0)),
            pl.BlockSpec((cin, c1), lambda t: (0, 0)),
        ],
        out_specs=[
            pl.BlockSpec((1, c1), lambda t: (0, 0)),
            pl.BlockSpec((1, c1), lambda t: (0, 0)),
        ],
        out_shape=[
            jax.ShapeDtypeStruct((1, c1), jnp.float32),
            jax.ShapeDtypeStruct((1, c1), jnp.float32),
        ],
    )(feat, counts3, wc)


def _pass_b(feat, csum3, wc, w2, aff1_a, aff1_b, a1):
    n_vox, cin = feat.shape
    nt = n_vox // VT
    c1 = wc.shape[1]
    c2 = w2.shape[1]

    def body(feat_ref, c_ref, wc_ref, w2_ref, a_ref, b_ref, s_ref,
             h2_ref, t1_ref, t2_ref):
        t = pl.program_id(0)
        hv = jnp.dot(feat_ref[...].astype(jnp.bfloat16),
                     wc_ref[...].astype(jnp.bfloat16),
                     preferred_element_type=jnp.float32)
        y = hv * a_ref[...] + b_ref[...]
        z = jnp.where(y >= 0, y, s_ref[0, 0] * y)
        h2 = jnp.dot(z.astype(jnp.bfloat16), w2_ref[...].astype(jnp.bfloat16),
                     preferred_element_type=jnp.float32)
        h2_ref[...] = h2
        c = c_ref[0]
        ch2 = c * h2
        p1 = jnp.sum(ch2, axis=0, keepdims=True)
        p2 = jnp.sum(ch2 * h2, axis=0, keepdims=True)

        @pl.when(t == 0)
        def _():
            t1_ref[...] = jnp.zeros_like(t1_ref)
            t2_ref[...] = jnp.zeros_like(t2_ref)

        t1_ref[...] += p1
        t2_ref[...] += p2

    return pl.pallas_call(
        body,
        grid=(nt,),
        in_specs=[
            pl.BlockSpec((VT, cin), lambda t: (t, 0)),
            pl.BlockSpec((1, VT, 1), lambda t: (t, 0, 0)),
            pl.BlockSpec((cin, c1), lambda t: (0, 0)),
            pl.BlockSpec((c1, c2), lambda t: (0, 0)),
            pl.BlockSpec((1, c1), lambda t: (0, 0)),
            pl.BlockSpec((1, c1), lambda t: (0, 0)),
            pl.BlockSpec((1, 1), lambda t: (0, 0)),
        ],
        out_specs=[
            pl.BlockSpec((VT, c2), lambda t: (t, 0)),
            pl.BlockSpec((1, c2), lambda t: (0, 0)),
            pl.BlockSpec((1, c2), lambda t: (0, 0)),
        ],
        out_shape=[
            jax.ShapeDtypeStruct((n_vox, c2), jnp.float32),
            jax.ShapeDtypeStruct((1, c2), jnp.float32),
            jax.ShapeDtypeStruct((1, c2), jnp.float32),
        ],
    )(feat, csum3, wc, w2, aff1_a, aff1_b, a1)


def _pass_c(h2, w3p, b3p, aff2_a, aff2_b, a2):
    n_vox, c2 = h2.shape
    nt = n_vox // VT
    d = w3p.shape[1]

    def body(h2_ref, w3_ref, b3_ref, a_ref, b_ref, s_ref, out_ref):
        y = h2_ref[...] * a_ref[...] + b_ref[...]
        z = jnp.where(y >= 0, y, s_ref[0, 0] * y)
        out_ref[...] = jnp.dot(z, w3_ref[...],
                               preferred_element_type=jnp.float32) + b3_ref[...]

    return pl.pallas_call(
        body,
        grid=(nt,),
        in_specs=[
            pl.BlockSpec((VT, c2), lambda t: (t, 0)),
            pl.BlockSpec((c2, d), lambda t: (0, 0)),
            pl.BlockSpec((1, d), lambda t: (0, 0)),
            pl.BlockSpec((1, c2), lambda t: (0, 0)),
            pl.BlockSpec((1, c2), lambda t: (0, 0)),
            pl.BlockSpec((1, 1), lambda t: (0, 0)),
        ],
        out_specs=pl.BlockSpec((VT, d), lambda t: (t, 0)),
        out_shape=jax.ShapeDtypeStruct((n_vox, d), jnp.float32),
    )(h2, w3p, b3p, aff2_a, aff2_b, a2)


def kernel(feat_voxel, xyz_voxel, v2p_v1, W_backbone, W1, g1, b1, a1,
           W2, g2, b2, a2, W3, b3):
    del xyz_voxel
    n_pts = v2p_v1.shape[0]
    n_vox = feat_voxel.shape[0]
    n_pts_f = jnp.float32(n_pts)

    wc = jnp.dot(W_backbone, W1, preferred_element_type=jnp.float32)
    d = W3.shape[1]
    b3p = b3.reshape(1, -1)

    hist = _histogram(v2p_v1, n_vox)
    counts = _merge_counts(hist, n_vox)
    counts3 = counts.reshape(n_vox // VT, VT, 1)

    s1, s2 = _stats1(feat_voxel, counts3, wc)
    mean1 = s1 / n_pts_f
    var1 = s2 / n_pts_f - mean1 * mean1
    aff1_a = g1.reshape(1, -1) / jnp.sqrt(var1 + EPS)
    aff1_b = b1.reshape(1, -1) - mean1 * aff1_a

    h2, t1, t2 = _pass_b(feat_voxel, counts3, wc, W2, aff1_a, aff1_b,
                         a1.reshape(1, 1))
    mean2 = t1 / n_pts_f
    var2 = t2 / n_pts_f - mean2 * mean2
    aff2_a = g2.reshape(1, -1) / jnp.sqrt(var2 + EPS)
    aff2_b = b2.reshape(1, -1) - mean2 * aff2_a

    table = _pass_c(h2, W3, b3p, aff2_a, aff2_b, a2.reshape(1, 1))

    lcm = G_CHUNK * d // math.gcd(G_CHUNK, d)
    n_pad = (n_pts * d + lcm - 1) // lcm * lcm // d - n_pts
    v2p_pad = jnp.pad(v2p_v1, (0, n_pad))
    idx_full = (v2p_pad[:, None] * d
                + jnp.arange(d, dtype=jnp.int32)[None, :]).reshape(-1)
    vals = _gather_elems(table.reshape(-1), idx_full.reshape(-1, 128))
    return vals[:n_pts * d].reshape(n_pts, d)


# speedup vs baseline: 1.1979x; 1.0437x over previous
import dataclasses
import functools

import jax
import jax.numpy as jnp
from jax import lax
from jax.experimental import pallas as pl
from jax.experimental.pallas import tpu as pltpu
from jax.experimental.pallas import tpu_sc as plsc

NC, NS = 2, 16
NW = NC * NS
H_ROWS = 16
Z_CHUNK = 2000
G_ROWS = 64
G_CHUNK = G_ROWS * 128
VT = 2000
EPS = 1e-5


def _sc_mesh():
    return plsc.VectorSubcoreMesh(core_axis_name="c", subcore_axis_name="s")


def _sc_no_layout_params():
    cp = pltpu.CompilerParams()
    if "needs_layout_passes" in pltpu.CompilerParams.__dataclass_fields__:
        cp = dataclasses.replace(cp, needs_layout_passes=False)
    return cp


def _histogram(v2p2d, n_vox):
    n_rows = v2p2d.shape[0]
    n_full = n_rows // H_ROWS
    tail_rows = n_rows - n_full * H_ROWS
    n_chunks = n_full + (1 if tail_rows else 0)
    chunks_per_w = (n_chunks + NW - 1) // NW
    nz = n_vox // Z_CHUNK
    z_per_tile = (nz + NS - 1) // NS

    @functools.partial(
        pl.kernel,
        out_type=jax.ShapeDtypeStruct((NC * n_vox,), jnp.float32),
        mesh=_sc_mesh(),
        scratch_types=[pltpu.VMEM_SHARED((n_vox,), jnp.float32),
                       pltpu.VMEM((H_ROWS, 128), jnp.int32),
                       pltpu.VMEM((Z_CHUNK,), jnp.float32)])
    def hist_kernel(idx_hbm, hist_hbm, counts_s, idx_v, buf_v):
        cid = lax.axis_index("c")
        sid = lax.axis_index("s")
        wid = sid * NC + cid

        @pl.loop(0, Z_CHUNK, step=16)
        def _(i):
            buf_v[pl.ds(i, 16)] = jnp.zeros((16,), jnp.float32)

        @pl.loop(0, z_per_tile)
        def _(j):
            k = sid + NS * j

            @pl.when(k < nz)
            def _():
                pltpu.sync_copy(buf_v, counts_s.at[pl.ds(k * Z_CHUNK, Z_CHUNK)])

        plsc.subcore_barrier()

        @pl.loop(0, 128, step=16)
        def _(i):
            buf_v[pl.ds(i, 16)] = jnp.ones((16,), jnp.float32)

        def do_chunk(row_base, rows):
            pltpu.sync_copy(idx_hbm.at[pl.ds(row_base, rows)],
                            idx_v.at[pl.ds(0, rows)])

            @pl.loop(0, rows)
            def _(r):
                pltpu.sync_copy(buf_v.at[pl.ds(0, 128)],
                                counts_s.at[idx_v.at[r]], add=True)

        @pl.loop(0, chunks_per_w)
        def _(j):
            c = wid + NW * j

            @pl.when(c < n_full)
            def _():
                do_chunk(c * H_ROWS, H_ROWS)

            if tail_rows:
                @pl.when(c == n_full)
                def _():
                    do_chunk(n_full * H_ROWS, tail_rows)

        plsc.subcore_barrier()

        @pl.loop(0, z_per_tile)
        def _(j):
            k = sid + NS * j

            @pl.when(k < nz)
            def _():
                pltpu.sync_copy(counts_s.at[pl.ds(k * Z_CHUNK, Z_CHUNK)],
                                buf_v)
                pltpu.sync_copy(
                    buf_v,
                    hist_hbm.at[pl.ds(cid * n_vox + k * Z_CHUNK, Z_CHUNK)])

    return hist_kernel(v2p2d)


def _gather_elems(table_flat, idx_full):
    n_elems = idx_full.shape[0]
    n_full = n_elems // G_CHUNK
    tail = n_elems - n_full * G_CHUNK
    tail_rows = tail // 128
    n_chunks = n_full + (1 if tail else 0)
    chunks_per_w = (n_chunks + NW - 1) // NW

    @functools.partial(
        pl.kernel,
        out_type=jax.ShapeDtypeStruct((n_elems,), jnp.float32),
        mesh=_sc_mesh(),
        scratch_types=[pltpu.VMEM((G_CHUNK,), jnp.int32),
                       pltpu.VMEM((G_CHUNK,), jnp.float32),
                       pltpu.SemaphoreType.DMA])
    def gather_kernel(table_hbm, idx_hbm, out_hbm, idx_v, vals_v, sem):
        wid = lax.axis_index("s") * NC + lax.axis_index("c")

        def do_chunk(base, rows):
            n = rows * 128
            pltpu.sync_copy(idx_hbm.at[pl.ds(base, n)], idx_v.at[pl.ds(0, n)])

            @pl.loop(0, rows)
            def _(r):
                pltpu.async_copy(
                    table_hbm.at[idx_v.at[pl.ds(r * 128, 128)]],
                    vals_v.at[pl.ds(r * 128, 128)], sem)

            @pl.loop(0, rows)
            def _(r):
                pltpu.make_async_copy(
                    table_hbm.at[idx_v.at[pl.ds(r * 128, 128)]],
                    vals_v.at[pl.ds(r * 128, 128)], sem).wait()

            pltpu.sync_copy(vals_v.at[pl.ds(0, n)], out_hbm.at[pl.ds(base, n)])

        @pl.loop(0, chunks_per_w)
        def _(j):
            c = wid + NW * j

            @pl.when(c < n_full)
            def _():
                do_chunk(c * G_CHUNK, G_ROWS)

            if tail:
                @pl.when(c == n_full)
                def _():
                    do_chunk(n_full * G_CHUNK, tail_rows)

    return gather_kernel(table_flat, idx_full)


def _stats1(feat, hist3d, wc):
    n_vox, cin = feat.shape
    nt = n_vox // VT
    c1 = wc.shape[1]

    def body(feat_ref, h_ref, wc_ref, s1_ref, s2_ref):
        t = pl.program_id(0)
        cnt = jnp.sum(h_ref[:, pl.ds(t, 1), :], axis=0)
        hv = jnp.dot(feat_ref[...].astype(jnp.bfloat16),
                     wc_ref[...].astype(jnp.bfloat16),
                     preferred_element_type=jnp.float32)
        p1 = jnp.dot(cnt, hv, preferred_element_type=jnp.float32)
        p2 = jnp.dot(cnt, hv * hv, preferred_element_type=jnp.float32)

        @pl.when(t == 0)
        def _():
            s1_ref[...] = jnp.zeros_like(s1_ref)
            s2_ref[...] = jnp.zeros_like(s2_ref)

        s1_ref[...] += p1
        s2_ref[...] += p2

    return pl.pallas_call(
        body,
        grid=(nt,),
        in_specs=[
            pl.BlockSpec((VT, cin), lambda t: (t, 0)),
            pl.BlockSpec((NC, nt, VT), lambda t: (0, 0, 0)),
            pl.BlockSpec((cin, c1), lambda t: (0, 0)),
        ],
        out_specs=[
            pl.BlockSpec((1, c1), lambda t: (0, 0)),
            pl.BlockSpec((1, c1), lambda t: (0, 0)),
        ],
        out_shape=[
            jax.ShapeDtypeStruct((1, c1), jnp.float32),
            jax.ShapeDtypeStruct((1, c1), jnp.float32),
        ],
    )(feat, hist3d, wc)


def _pass_b(feat, hist3d, wc, w2, aff1_a, aff1_b, a1):
    n_vox, cin = feat.shape
    nt = n_vox // VT
    c1 = wc.shape[1]
    c2 = w2.shape[1]

    def body(feat_ref, h_ref, wc_ref, w2_ref, a_ref, b_ref, s_ref,
             h2_ref, t1_ref, t2_ref):
        t = pl.program_id(0)
        cnt = jnp.sum(h_ref[:, pl.ds(t, 1), :], axis=0)
        hv = jnp.dot(feat_ref[...].astype(jnp.bfloat16),
                     wc_ref[...].astype(jnp.bfloat16),
                     preferred_element_type=jnp.float32)
        y = hv * a_ref[...] + b_ref[...]
        z = jnp.where(y >= 0, y, s_ref[0, 0] * y)
        h2 = jnp.dot(z.astype(jnp.bfloat16), w2_ref[...].astype(jnp.bfloat16),
                     preferred_element_type=jnp.float32)
        h2_ref[...] = h2
        p1 = jnp.dot(cnt, h2, preferred_element_type=jnp.float32)
        p2 = jnp.dot(cnt, h2 * h2, preferred_element_type=jnp.float32)

        @pl.when(t == 0)
        def _():
            t1_ref[...] = jnp.zeros_like(t1_ref)
            t2_ref[...] = jnp.zeros_like(t2_ref)

        t1_ref[...] += p1
        t2_ref[...] += p2

    return pl.pallas_call(
        body,
        grid=(nt,),
        in_specs=[
            pl.BlockSpec((VT, cin), lambda t: (t, 0)),
            pl.BlockSpec((NC, nt, VT), lambda t: (0, 0, 0)),
            pl.BlockSpec((cin, c1), lambda t: (0, 0)),
            pl.BlockSpec((c1, c2), lambda t: (0, 0)),
            pl.BlockSpec((1, c1), lambda t: (0, 0)),
            pl.BlockSpec((1, c1), lambda t: (0, 0)),
            pl.BlockSpec((1, 1), lambda t: (0, 0)),
        ],
        out_specs=[
            pl.BlockSpec((VT, c2), lambda t: (t, 0)),
            pl.BlockSpec((1, c2), lambda t: (0, 0)),
            pl.BlockSpec((1, c2), lambda t: (0, 0)),
        ],
        out_shape=[
            jax.ShapeDtypeStruct((n_vox, c2), jnp.float32),
            jax.ShapeDtypeStruct((1, c2), jnp.float32),
            jax.ShapeDtypeStruct((1, c2), jnp.float32),
        ],
    )(feat, hist3d, wc, w2, aff1_a, aff1_b, a1)


def _pass_c(h2, w3p, b3p, aff2_a, aff2_b, a2):
    n_vox, c2 = h2.shape
    nt = n_vox // VT
    d = w3p.shape[1]

    def body(h2_ref, w3_ref, b3_ref, a_ref, b_ref, s_ref, out_ref):
        y = h2_ref[...] * a_ref[...] + b_ref[...]
        z = jnp.where(y >= 0, y, s_ref[0, 0] * y)
        out_ref[...] = jnp.dot(z, w3_ref[...],
                               preferred_element_type=jnp.float32) + b3_ref[...]

    return pl.pallas_call(
        body,
        grid=(nt,),
        in_specs=[
            pl.BlockSpec((VT, c2), lambda t: (t, 0)),
            pl.BlockSpec((c2, d), lambda t: (0, 0)),
            pl.BlockSpec((1, d), lambda t: (0, 0)),
            pl.BlockSpec((1, c2), lambda t: (0, 0)),
            pl.BlockSpec((1, c2), lambda t: (0, 0)),
            pl.BlockSpec((1, 1), lambda t: (0, 0)),
        ],
        out_specs=pl.BlockSpec((VT, d), lambda t: (t, 0)),
        out_shape=jax.ShapeDtypeStruct((n_vox, d), jnp.float32),
    )(h2, w3p, b3p, aff2_a, aff2_b, a2)


def kernel(feat_voxel, xyz_voxel, v2p_v1, W_backbone, W1, g1, b1, a1,
           W2, g2, b2, a2, W3, b3):
    del xyz_voxel
    n_pts = v2p_v1.shape[0]
    n_vox = feat_voxel.shape[0]
    n_pts_f = jnp.float32(n_pts)

    wc = jnp.dot(W_backbone, W1, preferred_element_type=jnp.float32)
    d = W3.shape[1]
    b3p = b3.reshape(1, -1)

    hist = _histogram(v2p_v1.reshape(-1, 128), n_vox)
    hist3d = hist.reshape(NC, n_vox // VT, VT)

    s1, s2 = _stats1(feat_voxel, hist3d, wc)
    mean1 = s1 / n_pts_f
    var1 = s2 / n_pts_f - mean1 * mean1
    aff1_a = g1.reshape(1, -1) / jnp.sqrt(var1 + EPS)
    aff1_b = b1.reshape(1, -1) - mean1 * aff1_a

    h2, t1, t2 = _pass_b(feat_voxel, hist3d, wc, W2, aff1_a, aff1_b,
                         a1.reshape(1, 1))
    mean2 = t1 / n_pts_f
    var2 = t2 / n_pts_f - mean2 * mean2
    aff2_a = g2.reshape(1, -1) / jnp.sqrt(var2 + EPS)
    aff2_b = b2.reshape(1, -1) - mean2 * aff2_a

    table = _pass_c(h2, W3, b3p, aff2_a, aff2_b, a2.reshape(1, 1))

    idx_full = (v2p_v1[:, None] * d
                + jnp.arange(d, dtype=jnp.int32)[None, :]).reshape(-1)
    vals = _gather_elems(table.reshape(-1), idx_full)
    return vals.reshape(n_pts, d)


# speedup vs baseline: 3.5860x; 2.9936x over previous
import dataclasses
import functools

import jax
import jax.numpy as jnp
from jax import lax
from jax.experimental import pallas as pl
from jax.experimental.pallas import tpu as pltpu
from jax.experimental.pallas import tpu_sc as plsc

NC, NS = 2, 16
NW = NC * NS
H_ROWS = 16
Z_CHUNK = 2000
G_ROWS = 16
G_CHUNK = G_ROWS * 128
VT = 4000
VP = 100352
VTC = 2048
EPS = 1e-5


def _sc_mesh():
    return plsc.VectorSubcoreMesh(core_axis_name="c", subcore_axis_name="s")


def _sc_no_layout_params():
    cp = pltpu.CompilerParams()
    if "needs_layout_passes" in pltpu.CompilerParams.__dataclass_fields__:
        cp = dataclasses.replace(cp, needs_layout_passes=False)
    return cp


def _histogram(v2p2d, n_vox):
    n_rows = v2p2d.shape[0]
    n_full = n_rows // H_ROWS
    tail_rows = n_rows - n_full * H_ROWS
    n_chunks = n_full + (1 if tail_rows else 0)
    chunks_per_w = (n_chunks + NW - 1) // NW
    nz = n_vox // Z_CHUNK
    z_per_tile = (nz + NS - 1) // NS

    @functools.partial(
        pl.kernel,
        out_type=jax.ShapeDtypeStruct((NC * n_vox,), jnp.float32),
        mesh=_sc_mesh(),
        scratch_types=[pltpu.VMEM_SHARED((n_vox,), jnp.float32),
                       pltpu.VMEM((H_ROWS, 128), jnp.int32),
                       pltpu.VMEM((Z_CHUNK,), jnp.float32)])
    def hist_kernel(idx_hbm, hist_hbm, counts_s, idx_v, buf_v):
        cid = lax.axis_index("c")
        sid = lax.axis_index("s")
        wid = sid * NC + cid

        @pl.loop(0, Z_CHUNK, step=16)
        def _(i):
            buf_v[pl.ds(i, 16)] = jnp.zeros((16,), jnp.float32)

        @pl.loop(0, z_per_tile)
        def _(j):
            k = sid + NS * j

            @pl.when(k < nz)
            def _():
                pltpu.sync_copy(buf_v, counts_s.at[pl.ds(k * Z_CHUNK, Z_CHUNK)])

        plsc.subcore_barrier()

        @pl.loop(0, 128, step=16)
        def _(i):
            buf_v[pl.ds(i, 16)] = jnp.ones((16,), jnp.float32)

        def do_chunk(row_base, rows):
            pltpu.sync_copy(idx_hbm.at[pl.ds(row_base, rows)],
                            idx_v.at[pl.ds(0, rows)])

            @pl.loop(0, rows)
            def _(r):
                pltpu.sync_copy(buf_v.at[pl.ds(0, 128)],
                                counts_s.at[idx_v.at[r]], add=True)

        @pl.loop(0, chunks_per_w)
        def _(j):
            c = wid + NW * j

            @pl.when(c < n_full)
            def _():
                do_chunk(c * H_ROWS, H_ROWS)

            if tail_rows:
                @pl.when(c == n_full)
                def _():
                    do_chunk(n_full * H_ROWS, tail_rows)

        plsc.subcore_barrier()

        @pl.loop(0, z_per_tile)
        def _(j):
            k = sid + NS * j

            @pl.when(k < nz)
            def _():
                pltpu.sync_copy(counts_s.at[pl.ds(k * Z_CHUNK, Z_CHUNK)],
                                buf_v)
                pltpu.sync_copy(
                    buf_v,
                    hist_hbm.at[pl.ds(cid * n_vox + k * Z_CHUNK, Z_CHUNK)])

    return hist_kernel(v2p2d)


def _gather_planes(table_flat, v2p, d):
    n_pts = v2p.shape[0]
    n_vox = table_flat.shape[0] // d
    n_full = n_pts // G_CHUNK
    tail = n_pts - n_full * G_CHUNK
    tail_rows = tail // 128
    n_chunks = n_full + (1 if tail else 0)
    chunks_per_w = (n_chunks + NW - 1) // NW

    @functools.partial(
        pl.kernel,
        out_type=jax.ShapeDtypeStruct((d * n_pts,), jnp.float32),
        mesh=_sc_mesh(),
        scratch_types=[pltpu.VMEM((G_CHUNK,), jnp.int32),
                       pltpu.VMEM((G_CHUNK,), jnp.float32),
                       pltpu.SemaphoreType.DMA])
    def gather_kernel(table_hbm, idx_hbm, out_hbm, idx_v, vals_v, sem):
        wid = lax.axis_index("s") * NC + lax.axis_index("c")

        def do_chunk(base, rows):
            n = rows * 128
            pltpu.sync_copy(idx_hbm.at[pl.ds(base, n)], idx_v.at[pl.ds(0, n)])

            for c in range(d):
                plane = table_hbm.at[pl.ds(c * n_vox, n_vox)]

                @pl.loop(0, rows)
                def _(r):
                    pltpu.async_copy(
                        plane.at[idx_v.at[pl.ds(r * 128, 128)]],
                        vals_v.at[pl.ds(r * 128, 128)], sem)

                @pl.loop(0, rows)
                def _(r):
                    pltpu.make_async_copy(
                        plane.at[idx_v.at[pl.ds(r * 128, 128)]],
                        vals_v.at[pl.ds(r * 128, 128)], sem).wait()

                pltpu.sync_copy(vals_v.at[pl.ds(0, n)],
                                out_hbm.at[pl.ds(c * n_pts + base, n)])

        @pl.loop(0, chunks_per_w)
        def _(j):
            c = wid + NW * j

            @pl.when(c < n_full)
            def _():
                do_chunk(c * G_CHUNK, G_ROWS)

            if tail:
                @pl.when(c == n_full)
                def _():
                    do_chunk(n_full * G_CHUNK, tail_rows)

    return gather_kernel(table_flat, v2p)


def _stats1(feat, hist3d, wc):
    n_vox, cin = feat.shape
    nt = n_vox // VT
    c1 = wc.shape[1]

    def body(feat_ref, h_ref, wc_ref, s1_ref, s2_ref):
        t = pl.program_id(0)
        cnt = jnp.sum(h_ref[:, pl.ds(t, 1), :], axis=0)
        hv = jnp.dot(feat_ref[...].astype(jnp.bfloat16),
                     wc_ref[...].astype(jnp.bfloat16),
                     preferred_element_type=jnp.float32)
        p1 = jnp.dot(cnt, hv, preferred_element_type=jnp.float32)
        p2 = jnp.dot(cnt, hv * hv, preferred_element_type=jnp.float32)

        @pl.when(t == 0)
        def _():
            s1_ref[...] = jnp.zeros_like(s1_ref)
            s2_ref[...] = jnp.zeros_like(s2_ref)

        s1_ref[...] += p1
        s2_ref[...] += p2

    return pl.pallas_call(
        body,
        grid=(nt,),
        in_specs=[
            pl.BlockSpec((VT, cin), lambda t: (t, 0)),
            pl.BlockSpec((NC, nt, VT), lambda t: (0, 0, 0)),
            pl.BlockSpec((cin, c1), lambda t: (0, 0)),
        ],
        out_specs=[
            pl.BlockSpec((1, c1), lambda t: (0, 0)),
            pl.BlockSpec((1, c1), lambda t: (0, 0)),
        ],
        out_shape=[
            jax.ShapeDtypeStruct((1, c1), jnp.float32),
            jax.ShapeDtypeStruct((1, c1), jnp.float32),
        ],
    )(feat, hist3d, wc)


def _pass_b(feat, hist3d, wc, w2, aff1_a, aff1_b, a1):
    n_vox, cin = feat.shape
    nt = n_vox // VT
    c1 = wc.shape[1]
    c2 = w2.shape[1]

    def body(feat_ref, h_ref, wc_ref, w2_ref, a_ref, b_ref, s_ref,
             h2_ref, t1_ref, t2_ref):
        t = pl.program_id(0)
        cnt = jnp.sum(h_ref[:, pl.ds(t, 1), :], axis=0)
        hv = jnp.dot(feat_ref[...].astype(jnp.bfloat16),
                     wc_ref[...].astype(jnp.bfloat16),
                     preferred_element_type=jnp.float32)
        y = hv * a_ref[...] + b_ref[...]
        z = jnp.where(y >= 0, y, s_ref[0, 0] * y)
        h2 = jnp.dot(z.astype(jnp.bfloat16), w2_ref[...].astype(jnp.bfloat16),
                     preferred_element_type=jnp.float32)
        h2_ref[...] = h2
        p1 = jnp.dot(cnt, h2, preferred_element_type=jnp.float32)
        p2 = jnp.dot(cnt, h2 * h2, preferred_element_type=jnp.float32)

        @pl.when(t == 0)
        def _():
            t1_ref[...] = jnp.zeros_like(t1_ref)
            t2_ref[...] = jnp.zeros_like(t2_ref)

        t1_ref[...] += p1
        t2_ref[...] += p2

    return pl.pallas_call(
        body,
        grid=(nt,),
        in_specs=[
            pl.BlockSpec((VT, cin), lambda t: (t, 0)),
            pl.BlockSpec((NC, nt, VT), lambda t: (0, 0, 0)),
            pl.BlockSpec((cin, c1), lambda t: (0, 0)),
            pl.BlockSpec((c1, c2), lambda t: (0, 0)),
            pl.BlockSpec((1, c1), lambda t: (0, 0)),
            pl.BlockSpec((1, c1), lambda t: (0, 0)),
            pl.BlockSpec((1, 1), lambda t: (0, 0)),
        ],
        out_specs=[
            pl.BlockSpec((VT, c2), lambda t: (t, 0)),
            pl.BlockSpec((1, c2), lambda t: (0, 0)),
            pl.BlockSpec((1, c2), lambda t: (0, 0)),
        ],
        out_shape=[
            jax.ShapeDtypeStruct((VP, c2), jnp.float32),
            jax.ShapeDtypeStruct((1, c2), jnp.float32),
            jax.ShapeDtypeStruct((1, c2), jnp.float32),
        ],
    )(feat, hist3d, wc, w2, aff1_a, aff1_b, a1)


def _pass_c(h2, w3, b3, aff2_a, aff2_b, a2):
    vp, c2 = h2.shape
    nt = vp // VTC
    d = w3.shape[1]

    def body(h2_ref, w3_ref, b3_ref, a_ref, b_ref, s_ref, out_ref):
        y = h2_ref[...] * a_ref[...] + b_ref[...]
        z = jnp.where(y >= 0, y, s_ref[0, 0] * y)
        zt = lax.dot_general(w3_ref[...], z, (((0,), (1,)), ((), ())),
                             preferred_element_type=jnp.float32)
        out_ref[...] = zt + b3_ref[...]

    return pl.pallas_call(
        body,
        grid=(nt,),
        in_specs=[
            pl.BlockSpec((VTC, c2), lambda t: (t, 0)),
            pl.BlockSpec((c2, d), lambda t: (0, 0)),
            pl.BlockSpec((d, 1), lambda t: (0, 0)),
            pl.BlockSpec((1, c2), lambda t: (0, 0)),
            pl.BlockSpec((1, c2), lambda t: (0, 0)),
            pl.BlockSpec((1, 1), lambda t: (0, 0)),
        ],
        out_specs=pl.BlockSpec((d, VTC), lambda t: (0, t)),
        out_shape=jax.ShapeDtypeStruct((d, vp), jnp.float32),
    )(h2, w3, b3, aff2_a, aff2_b, a2)


def kernel(feat_voxel, xyz_voxel, v2p_v1, W_backbone, W1, g1, b1, a1,
           W2, g2, b2, a2, W3, b3):
    del xyz_voxel
    n_pts = v2p_v1.shape[0]
    n_vox = feat_voxel.shape[0]
    n_pts_f = jnp.float32(n_pts)

    wc = jnp.dot(W_backbone, W1,
                 preferred_element_type=jnp.float32).astype(jnp.bfloat16)
    feat16 = feat_voxel.astype(jnp.bfloat16)
    w2_16 = W2.astype(jnp.bfloat16)
    d = W3.shape[1]
    b3p = b3.reshape(-1, 1)

    hist = _histogram(v2p_v1.reshape(-1, 128), n_vox)
    hist3d = hist.reshape(NC, n_vox // VT, VT)

    s1, s2 = _stats1(feat16, hist3d, wc)
    mean1 = s1 / n_pts_f
    var1 = s2 / n_pts_f - mean1 * mean1
    aff1_a = g1.reshape(1, -1) / jnp.sqrt(var1 + EPS)
    aff1_b = b1.reshape(1, -1) - mean1 * aff1_a

    h2, t1, t2 = _pass_b(feat16, hist3d, wc, w2_16, aff1_a, aff1_b,
                         a1.reshape(1, 1))
    mean2 = t1 / n_pts_f
    var2 = t2 / n_pts_f - mean2 * mean2
    aff2_a = g2.reshape(1, -1) / jnp.sqrt(var2 + EPS)
    aff2_b = b2.reshape(1, -1) - mean2 * aff2_a

    table_t = _pass_c(h2, W3, b3p, aff2_a, aff2_b, a2.reshape(1, 1))

    out_t = _gather_planes(table_t.reshape(-1), v2p_v1, d)
    return out_t.reshape(d, n_pts).T


# speedup vs baseline: 4.8979x; 1.3658x over previous
import dataclasses
import functools

import jax
import jax.numpy as jnp
from jax import lax
from jax.experimental import pallas as pl
from jax.experimental.pallas import tpu as pltpu
from jax.experimental.pallas import tpu_sc as plsc

NC, NS = 2, 16
NW = NC * NS
H_ROWS = 16
Z_CHUNK = 2000
G_ROWS = 16
G_CHUNK = G_ROWS * 128
VT = 4000
VP = 100352
VTC = 2048
EPS = 1e-5


def _sc_mesh():
    return plsc.VectorSubcoreMesh(core_axis_name="c", subcore_axis_name="s")


def _sc_no_layout_params():
    cp = pltpu.CompilerParams()
    if "needs_layout_passes" in pltpu.CompilerParams.__dataclass_fields__:
        cp = dataclasses.replace(cp, needs_layout_passes=False)
    return cp


def _histogram(v2p2d, n_vox):
    n_rows = v2p2d.shape[0]
    n_full = n_rows // H_ROWS
    tail_rows = n_rows - n_full * H_ROWS
    n_chunks = n_full + (1 if tail_rows else 0)
    chunks_per_w = (n_chunks + NW - 1) // NW
    nz = n_vox // Z_CHUNK
    z_per_tile = (nz + NS - 1) // NS

    @functools.partial(
        pl.kernel,
        out_type=jax.ShapeDtypeStruct((NC * n_vox,), jnp.float32),
        mesh=_sc_mesh(),
        scratch_types=[pltpu.VMEM_SHARED((n_vox,), jnp.float32),
                       pltpu.VMEM((H_ROWS, 128), jnp.int32),
                       pltpu.VMEM((Z_CHUNK,), jnp.float32)])
    def hist_kernel(idx_hbm, hist_hbm, counts_s, idx_v, buf_v):
        cid = lax.axis_index("c")
        sid = lax.axis_index("s")
        wid = sid * NC + cid

        @pl.loop(0, Z_CHUNK, step=16)
        def _(i):
            buf_v[pl.ds(i, 16)] = jnp.zeros((16,), jnp.float32)

        @pl.loop(0, z_per_tile)
        def _(j):
            k = sid + NS * j

            @pl.when(k < nz)
            def _():
                pltpu.sync_copy(buf_v, counts_s.at[pl.ds(k * Z_CHUNK, Z_CHUNK)])

        plsc.subcore_barrier()

        @pl.loop(0, 128, step=16)
        def _(i):
            buf_v[pl.ds(i, 16)] = jnp.ones((16,), jnp.float32)

        def do_chunk(row_base, rows):
            pltpu.sync_copy(idx_hbm.at[pl.ds(row_base, rows)],
                            idx_v.at[pl.ds(0, rows)])

            @pl.loop(0, rows)
            def _(r):
                pltpu.sync_copy(buf_v.at[pl.ds(0, 128)],
                                counts_s.at[idx_v.at[r]], add=True)

        @pl.loop(0, chunks_per_w)
        def _(j):
            c = wid + NW * j

            @pl.when(c < n_full)
            def _():
                do_chunk(c * H_ROWS, H_ROWS)

            if tail_rows:
                @pl.when(c == n_full)
                def _():
                    do_chunk(n_full * H_ROWS, tail_rows)

        plsc.subcore_barrier()

        @pl.loop(0, z_per_tile)
        def _(j):
            k = sid + NS * j

            @pl.when(k < nz)
            def _():
                pltpu.sync_copy(counts_s.at[pl.ds(k * Z_CHUNK, Z_CHUNK)],
                                buf_v)
                pltpu.sync_copy(
                    buf_v,
                    hist_hbm.at[pl.ds(cid * n_vox + k * Z_CHUNK, Z_CHUNK)])

    return hist_kernel(v2p2d)


def _gather_planes(table_flat, v2p, d):
    n_pts = v2p.shape[0]
    n_vox = table_flat.shape[0] // d
    n_full = n_pts // G_CHUNK
    tail = n_pts - n_full * G_CHUNK
    tail_rows = tail // 128
    n_chunks = n_full + (1 if tail else 0)
    chunks_per_w = (n_chunks + NW - 1) // NW
    tbl = d * n_vox
    cp_full = tbl // G_CHUNK
    cp_tail = tbl - cp_full * G_CHUNK
    n_cp = cp_full + (1 if cp_tail else 0)
    cp_per_s = (n_cp + NS - 1) // NS

    @functools.partial(
        pl.kernel,
        out_type=jax.ShapeDtypeStruct((d * n_pts,), jnp.float32),
        mesh=_sc_mesh(),
        scratch_types=[pltpu.VMEM((G_CHUNK,), jnp.int32),
                       pltpu.VMEM((G_CHUNK,), jnp.float32),
                       pltpu.VMEM_SHARED((d * n_vox,), jnp.float32),
                       pltpu.SemaphoreType.DMA])
    def gather_kernel(table_hbm, idx_hbm, out_hbm, idx_v, vals_v, table_s,
                      sem):
        sid = lax.axis_index("s")
        wid = sid * NC + lax.axis_index("c")

        def cp_in(base, n):
            pltpu.sync_copy(table_hbm.at[pl.ds(base, n)],
                            vals_v.at[pl.ds(0, n)])
            pltpu.sync_copy(vals_v.at[pl.ds(0, n)],
                            table_s.at[pl.ds(base, n)])

        @pl.loop(0, cp_per_s)
        def _(j):
            k = sid + NS * j

            @pl.when(k < cp_full)
            def _():
                cp_in(k * G_CHUNK, G_CHUNK)

            if cp_tail:
                @pl.when(k == cp_full)
                def _():
                    cp_in(cp_full * G_CHUNK, cp_tail)

        plsc.subcore_barrier()

        def do_chunk(base, rows):
            n = rows * 128
            pltpu.sync_copy(idx_hbm.at[pl.ds(base, n)], idx_v.at[pl.ds(0, n)])

            for c in range(d):
                plane = table_s.at[pl.ds(c * n_vox, n_vox)]

                @pl.loop(0, rows)
                def _(r):
                    pltpu.async_copy(
                        plane.at[idx_v.at[pl.ds(r * 128, 128)]],
                        vals_v.at[pl.ds(r * 128, 128)], sem)

                @pl.loop(0, rows)
                def _(r):
                    pltpu.make_async_copy(
                        plane.at[idx_v.at[pl.ds(r * 128, 128)]],
                        vals_v.at[pl.ds(r * 128, 128)], sem).wait()

                pltpu.sync_copy(vals_v.at[pl.ds(0, n)],
                                out_hbm.at[pl.ds(c * n_pts + base, n)])

        @pl.loop(0, chunks_per_w)
        def _(j):
            c = wid + NW * j

            @pl.when(c < n_full)
            def _():
                do_chunk(c * G_CHUNK, G_ROWS)

            if tail:
                @pl.when(c == n_full)
                def _():
                    do_chunk(n_full * G_CHUNK, tail_rows)

    return gather_kernel(table_flat, v2p)


def _stats1(feat, hist3d, wc):
    n_vox, cin = feat.shape
    nt = n_vox // VT
    c1 = wc.shape[1]

    def body(feat_ref, h_ref, wc_ref, s1_ref, s2_ref):
        t = pl.program_id(0)
        cnt = jnp.sum(h_ref[:, pl.ds(t, 1), :], axis=0)
        hv = jnp.dot(feat_ref[...].astype(jnp.bfloat16),
                     wc_ref[...].astype(jnp.bfloat16),
                     preferred_element_type=jnp.float32)
        p1 = jnp.dot(cnt, hv, preferred_element_type=jnp.float32)
        p2 = jnp.dot(cnt, hv * hv, preferred_element_type=jnp.float32)

        @pl.when(t == 0)
        def _():
            s1_ref[...] = jnp.zeros_like(s1_ref)
            s2_ref[...] = jnp.zeros_like(s2_ref)

        s1_ref[...] += p1
        s2_ref[...] += p2

    return pl.pallas_call(
        body,
        grid=(nt,),
        in_specs=[
            pl.BlockSpec((VT, cin), lambda t: (t, 0)),
            pl.BlockSpec((NC, nt, VT), lambda t: (0, 0, 0)),
            pl.BlockSpec((cin, c1), lambda t: (0, 0)),
        ],
        out_specs=[
            pl.BlockSpec((1, c1), lambda t: (0, 0)),
            pl.BlockSpec((1, c1), lambda t: (0, 0)),
        ],
        out_shape=[
            jax.ShapeDtypeStruct((1, c1), jnp.float32),
            jax.ShapeDtypeStruct((1, c1), jnp.float32),
        ],
    )(feat, hist3d, wc)


def _pass_b(feat, hist3d, wc, w2, aff1_a, aff1_b, a1):
    n_vox, cin = feat.shape
    nt = n_vox // VT
    c1 = wc.shape[1]
    c2 = w2.shape[1]

    def body(feat_ref, h_ref, wc_ref, w2_ref, a_ref, b_ref, s_ref,
             h2_ref, t1_ref, t2_ref):
        t = pl.program_id(0)
        cnt = jnp.sum(h_ref[:, pl.ds(t, 1), :], axis=0)
        hv = jnp.dot(feat_ref[...].astype(jnp.bfloat16),
                     wc_ref[...].astype(jnp.bfloat16),
                     preferred_element_type=jnp.float32)
        y = hv * a_ref[...] + b_ref[...]
        z = jnp.where(y >= 0, y, s_ref[0, 0] * y)
        h2 = jnp.dot(z.astype(jnp.bfloat16), w2_ref[...].astype(jnp.bfloat16),
                     preferred_element_type=jnp.float32)
        h2_ref[...] = h2
        p1 = jnp.dot(cnt, h2, preferred_element_type=jnp.float32)
        p2 = jnp.dot(cnt, h2 * h2, preferred_element_type=jnp.float32)

        @pl.when(t == 0)
        def _():
            t1_ref[...] = jnp.zeros_like(t1_ref)
            t2_ref[...] = jnp.zeros_like(t2_ref)

        t1_ref[...] += p1
        t2_ref[...] += p2

    return pl.pallas_call(
        body,
        grid=(nt,),
        in_specs=[
            pl.BlockSpec((VT, cin), lambda t: (t, 0)),
            pl.BlockSpec((NC, nt, VT), lambda t: (0, 0, 0)),
            pl.BlockSpec((cin, c1), lambda t: (0, 0)),
            pl.BlockSpec((c1, c2), lambda t: (0, 0)),
            pl.BlockSpec((1, c1), lambda t: (0, 0)),
            pl.BlockSpec((1, c1), lambda t: (0, 0)),
            pl.BlockSpec((1, 1), lambda t: (0, 0)),
        ],
        out_specs=[
            pl.BlockSpec((VT, c2), lambda t: (t, 0)),
            pl.BlockSpec((1, c2), lambda t: (0, 0)),
            pl.BlockSpec((1, c2), lambda t: (0, 0)),
        ],
        out_shape=[
            jax.ShapeDtypeStruct((VP, c2), jnp.float32),
            jax.ShapeDtypeStruct((1, c2), jnp.float32),
            jax.ShapeDtypeStruct((1, c2), jnp.float32),
        ],
    )(feat, hist3d, wc, w2, aff1_a, aff1_b, a1)


def _pass_c(h2, w3, b3, aff2_a, aff2_b, a2):
    vp, c2 = h2.shape
    nt = vp // VTC
    d = w3.shape[1]

    def body(h2_ref, w3_ref, b3_ref, a_ref, b_ref, s_ref, out_ref):
        y = h2_ref[...] * a_ref[...] + b_ref[...]
        z = jnp.where(y >= 0, y, s_ref[0, 0] * y)
        zt = lax.dot_general(w3_ref[...], z, (((0,), (1,)), ((), ())),
                             preferred_element_type=jnp.float32)
        out_ref[...] = zt + b3_ref[...]

    return pl.pallas_call(
        body,
        grid=(nt,),
        in_specs=[
            pl.BlockSpec((VTC, c2), lambda t: (t, 0)),
            pl.BlockSpec((c2, d), lambda t: (0, 0)),
            pl.BlockSpec((d, 1), lambda t: (0, 0)),
            pl.BlockSpec((1, c2), lambda t: (0, 0)),
            pl.BlockSpec((1, c2), lambda t: (0, 0)),
            pl.BlockSpec((1, 1), lambda t: (0, 0)),
        ],
        out_specs=pl.BlockSpec((d, VTC), lambda t: (0, t)),
        out_shape=jax.ShapeDtypeStruct((d, vp), jnp.float32),
    )(h2, w3, b3, aff2_a, aff2_b, a2)


def kernel(feat_voxel, xyz_voxel, v2p_v1, W_backbone, W1, g1, b1, a1,
           W2, g2, b2, a2, W3, b3):
    del xyz_voxel
    n_pts = v2p_v1.shape[0]
    n_vox = feat_voxel.shape[0]
    n_pts_f = jnp.float32(n_pts)

    wc = jnp.dot(W_backbone, W1,
                 preferred_element_type=jnp.float32).astype(jnp.bfloat16)
    feat16 = feat_voxel.astype(jnp.bfloat16)
    w2_16 = W2.astype(jnp.bfloat16)
    d = W3.shape[1]
    b3p = b3.reshape(-1, 1)

    hist = _histogram(v2p_v1.reshape(-1, 128), n_vox)
    hist3d = hist.reshape(NC, n_vox // VT, VT)

    s1, s2 = _stats1(feat16, hist3d, wc)
    mean1 = s1 / n_pts_f
    var1 = s2 / n_pts_f - mean1 * mean1
    aff1_a = g1.reshape(1, -1) / jnp.sqrt(var1 + EPS)
    aff1_b = b1.reshape(1, -1) - mean1 * aff1_a

    h2, t1, t2 = _pass_b(feat16, hist3d, wc, w2_16, aff1_a, aff1_b,
                         a1.reshape(1, 1))
    mean2 = t1 / n_pts_f
    var2 = t2 / n_pts_f - mean2 * mean2
    aff2_a = g2.reshape(1, -1) / jnp.sqrt(var2 + EPS)
    aff2_b = b2.reshape(1, -1) - mean2 * aff2_a

    table_t = _pass_c(h2, W3, b3p, aff2_a, aff2_b, a2.reshape(1, 1))

    out_t = _gather_planes(table_t.reshape(-1), v2p_v1, d)
    return out_t.reshape(d, n_pts).T
